# baseline (speedup 1.0000x reference)
import jax
import jax.numpy as jnp
from jax.experimental import pallas as pl

_IMG_SIZE = 512
_NUM_ANCHORS = 3
_NUM_CLASSES = 80
_ATTRS = 5 + _NUM_CLASSES
_ANCHORS_W = (10.0, 16.0, 33.0)
_ANCHORS_H = (13.0, 30.0, 23.0)


def _decode_body(x_ref, o_ref):
    a = pl.program_id(1)
    v = x_ref[0]
    h, w = v.shape[1], v.shape[2]
    stride = float(_IMG_SIZE) / float(h)
    sig = jax.nn.sigmoid(v)
    gx = jax.lax.broadcasted_iota(jnp.int32, (1, h, w), 2).astype(jnp.float32)
    gy = jax.lax.broadcasted_iota(jnp.int32, (1, h, w), 1).astype(jnp.float32)
    aw = jnp.where(a == 0, _ANCHORS_W[0], jnp.where(a == 1, _ANCHORS_W[1], _ANCHORS_W[2]))
    ah = jnp.where(a == 0, _ANCHORS_H[0], jnp.where(a == 1, _ANCHORS_H[1], _ANCHORS_H[2]))
    res = jnp.concatenate(
        [
            (sig[0:1] + gx) * stride,
            (sig[1:2] + gy) * stride,
            jnp.exp(v[2:3]) * aw,
            jnp.exp(v[3:4]) * ah,
            sig[4:],
        ],
        axis=0,
    )
    o_ref[0] = jnp.transpose(res, (1, 2, 0)).reshape(h * w, _ATTRS)


def kernel(input):
    bs, c, in_h, in_w = input.shape
    hw = in_h * in_w
    out = pl.pallas_call(
        _decode_body,
        grid=(bs, _NUM_ANCHORS),
        in_specs=[pl.BlockSpec((1, _ATTRS, in_h, in_w), lambda b, a: (b, a, 0, 0))],
        out_specs=pl.BlockSpec((1, hw, _ATTRS), lambda b, a: (b, a, 0)),
        out_shape=jax.ShapeDtypeStruct((bs, _NUM_ANCHORS * hw, _ATTRS), jnp.float32),
    )(input)
    return out


# speedup vs baseline: 1.0577x; 1.0577x over previous
import jax
import jax.numpy as jnp
from jax.experimental import pallas as pl

_IMG_SIZE = 512
_NUM_ANCHORS = 3
_NUM_CLASSES = 80
_ATTRS = 5 + _NUM_CLASSES
_ANCHORS_W = (10.0, 16.0, 33.0)
_ANCHORS_H = (13.0, 30.0, 23.0)


def _decode_body(x_ref, o_ref):
    v = x_ref[0]
    h, w = v.shape[1], v.shape[2]
    stride = float(_IMG_SIZE) / float(h)
    sig = jax.nn.sigmoid(v)
    gx = jax.lax.broadcasted_iota(jnp.int32, (1, h, w), 2).astype(jnp.float32)
    gy = jax.lax.broadcasted_iota(jnp.int32, (1, h, w), 1).astype(jnp.float32)
    parts = []
    for a in range(_NUM_ANCHORS):
        base = a * _ATTRS
        parts.append((sig[base : base + 1] + gx) * stride)
        parts.append((sig[base + 1 : base + 2] + gy) * stride)
        parts.append(jnp.exp(v[base + 2 : base + 3]) * _ANCHORS_W[a])
        parts.append(jnp.exp(v[base + 3 : base + 4]) * _ANCHORS_H[a])
        parts.append(sig[base + 4 : base + _ATTRS])
    res = jnp.concatenate(parts, axis=0)
    r4 = res.reshape(_NUM_ANCHORS, _ATTRS, h, w)
    o_ref[0] = jnp.transpose(r4, (0, 2, 3, 1)).reshape(_NUM_ANCHORS * h * w, _ATTRS)


def kernel(input):
    bs, c, in_h, in_w = input.shape
    hw = in_h * in_w
    out = pl.pallas_call(
        _decode_body,
        grid=(bs,),
        in_specs=[pl.BlockSpec((1, c, in_h, in_w), lambda b: (b, 0, 0, 0))],
        out_specs=pl.BlockSpec((1, _NUM_ANCHORS * hw, _ATTRS), lambda b: (b, 0, 0)),
        out_shape=jax.ShapeDtypeStruct((bs, _NUM_ANCHORS * hw, _ATTRS), jnp.float32),
    )(input)
    return out
